# baseline (speedup 1.0000x reference)
import jax
import jax.numpy as jnp
from jax.experimental import pallas as pl
from jax.experimental.pallas import tpu as pltpu

_N = 2048
_D = 128
_P = 8
_TOPK = 32
_PD = _P * _D
_BLK = 256
_GRID = _N // _BLK
_COARSE_ITERS = 8
_FINE_ITERS = 14


def _fused_kernel(f_ref, w_ref, o_ref, y_s, yt_s, s_s, tc_s, tr_s):
    pid = pl.program_id(0)

    @pl.when(pid < _GRID)
    def _prep():
        f = f_ref[...]
        w = w_ref[...]
        cols = []
        for p in range(_P):
            fw = f * w[p:p + 1, :]
            n = jnp.sqrt(jnp.sum(fw * fw, axis=1, keepdims=True))
            cols.append(fw / jnp.maximum(n, 1e-12))
        y = jnp.concatenate(cols, axis=1)
        row = pid * _BLK
        y_s[pl.ds(row, _BLK), :] = y
        yt_s[:, pl.ds(row, _BLK)] = y.T

    @pl.when((pid >= _GRID) & (pid < 2 * _GRID))
    def _sim():
        row = (pid - _GRID) * _BLK
        y = y_s[pl.ds(row, _BLK), :]
        s = jax.lax.dot_general(
            y, yt_s[...], (((1,), (0,)), ((), ())),
            preferred_element_type=jnp.float32) * (1.0 / _P)
        s_s[pl.ds(row, _BLK), :] = s

        sb = s.astype(jnp.bfloat16)

        def cbody(_, carry):
            lo, hi = carry
            mid = (lo + hi) * 0.5
            cnt = jnp.count_nonzero(sb >= mid.astype(jnp.bfloat16),
                                    axis=1, keepdims=True)
            pred = cnt >= _TOPK
            return jnp.where(pred, mid, lo), jnp.where(pred, hi, mid)

        lo, hi = jax.lax.fori_loop(
            0, _COARSE_ITERS, cbody,
            (jnp.full((_BLK, 1), -1.25, jnp.float32),
             jnp.full((_BLK, 1), 1.25, jnp.float32)),
            unroll=_COARSE_ITERS)
        lo = lo - 0.005
        hi = hi + 0.005

        def body(_, carry):
            lo, hi = carry
            mid = (lo + hi) * 0.5
            cnt = jnp.count_nonzero(s >= mid, axis=1, keepdims=True)
            pred = cnt >= _TOPK
            return jnp.where(pred, mid, lo), jnp.where(pred, hi, mid)

        lo, hi = jax.lax.fori_loop(
            0, _FINE_ITERS, body, (lo, hi), unroll=_FINE_ITERS)
        tc_s[pl.ds(row, _BLK), :] = lo
        tr_s[:, pl.ds(row, _BLK)] = lo.T

    @pl.when(pid >= 2 * _GRID)
    def _mask():
        row = (pid - 2 * _GRID) * _BLK
        s = s_s[pl.ds(row, _BLK), :]
        ti = tc_s[pl.ds(row, _BLK), :]
        tj = tr_s[...]
        keep = (s >= ti).astype(jnp.float32) + (s >= tj).astype(jnp.float32)
        o_ref[...] = s * keep * 0.5


@jax.jit
def kernel(features, weight_tensor):
    return pl.pallas_call(
        _fused_kernel,
        grid=(3 * _GRID,),
        in_specs=[
            pl.BlockSpec((_BLK, _D), lambda i: (jnp.minimum(i, _GRID - 1), 0)),
            pl.BlockSpec((_P, _D), lambda i: (0, 0)),
        ],
        out_specs=pl.BlockSpec(
            (_BLK, _N), lambda i: (jnp.maximum(i - 2 * _GRID, 0), 0)),
        out_shape=jax.ShapeDtypeStruct((_N, _N), jnp.float32),
        scratch_shapes=[
            pltpu.VMEM((_N, _PD), jnp.float32),
            pltpu.VMEM((_PD, _N), jnp.float32),
            pltpu.VMEM((_N, _N), jnp.float32),
            pltpu.VMEM((_N, 1), jnp.float32),
            pltpu.VMEM((1, _N), jnp.float32),
        ],
    )(features, weight_tensor)


# speedup vs baseline: 1.2842x; 1.2842x over previous
import jax
import jax.numpy as jnp
from jax.experimental import pallas as pl
from jax.experimental.pallas import tpu as pltpu

_N = 2048
_D = 128
_P = 8
_TOPK = 32
_PD = _P * _D
_BLK = 512
_GRID = _N // _BLK
_BISECT_ITERS = 21


def _fused_kernel(f_ref, w_ref, o_ref, y_s, yt_s, s_s, tc_s, tr_s):
    pid = pl.program_id(0)

    @pl.when(pid < _GRID)
    def _prep():
        f = f_ref[...]
        w = w_ref[...]
        cols = []
        for p in range(_P):
            fw = f * w[p:p + 1, :]
            n = jnp.sqrt(jnp.sum(fw * fw, axis=1, keepdims=True))
            cols.append(fw / jnp.maximum(n, 1e-12))
        y = jnp.concatenate(cols, axis=1)
        row = pid * _BLK
        y_s[pl.ds(row, _BLK), :] = y
        yt_s[:, pl.ds(row, _BLK)] = y.T

    @pl.when((pid >= _GRID) & (pid < 2 * _GRID))
    def _sim():
        row = (pid - _GRID) * _BLK
        y = y_s[pl.ds(row, _BLK), :]
        s = jax.lax.dot_general(
            y, yt_s[...], (((1,), (0,)), ((), ())),
            preferred_element_type=jnp.float32) * (1.0 / _P)
        s_s[pl.ds(row, _BLK), :] = s

        def body(_, carry):
            lo, hi = carry
            mid = (lo + hi) * 0.5
            cnt = jnp.count_nonzero(s >= mid, axis=1, keepdims=True)
            pred = cnt >= _TOPK
            return jnp.where(pred, mid, lo), jnp.where(pred, hi, mid)

        lo, hi = jax.lax.fori_loop(
            0, _BISECT_ITERS, body,
            (jnp.full((_BLK, 1), -1.25, jnp.float32),
             jnp.full((_BLK, 1), 1.25, jnp.float32)),
            unroll=_BISECT_ITERS)
        tc_s[pl.ds(row, _BLK), :] = lo
        tr_s[:, pl.ds(row, _BLK)] = lo.T

    @pl.when(pid >= 2 * _GRID)
    def _mask():
        row = (pid - 2 * _GRID) * _BLK
        s = s_s[pl.ds(row, _BLK), :]
        ti = tc_s[pl.ds(row, _BLK), :]
        tj = tr_s[...]
        keep = (s >= ti).astype(jnp.float32) + (s >= tj).astype(jnp.float32)
        o_ref[...] = s * keep * 0.5


@jax.jit
def kernel(features, weight_tensor):
    return pl.pallas_call(
        _fused_kernel,
        grid=(3 * _GRID,),
        in_specs=[
            pl.BlockSpec((_BLK, _D), lambda i: (jnp.minimum(i, _GRID - 1), 0)),
            pl.BlockSpec((_P, _D), lambda i: (0, 0)),
        ],
        out_specs=pl.BlockSpec(
            (_BLK, _N), lambda i: (jnp.maximum(i - 2 * _GRID, 0), 0)),
        out_shape=jax.ShapeDtypeStruct((_N, _N), jnp.float32),
        scratch_shapes=[
            pltpu.VMEM((_N, _PD), jnp.float32),
            pltpu.VMEM((_PD, _N), jnp.float32),
            pltpu.VMEM((_N, _N), jnp.float32),
            pltpu.VMEM((_N, 1), jnp.float32),
            pltpu.VMEM((1, _N), jnp.float32),
        ],
    )(features, weight_tensor)


# speedup vs baseline: 1.2851x; 1.0007x over previous
import jax
import jax.numpy as jnp
from jax.experimental import pallas as pl
from jax.experimental.pallas import tpu as pltpu

_N = 2048
_D = 128
_P = 8
_TOPK = 32
_PD = _P * _D
_BLK = 512
_GRID = _N // _BLK
_BISECT_ITERS = 21


def _fused_kernel(f_ref, w_ref, o_ref, y_s, yt_s, s_s, tc_s, tr_s):
    pid = pl.program_id(0)

    @pl.when(pid < _GRID)
    def _prep():
        f = f_ref[...]
        w = w_ref[...]
        cols = []
        for p in range(_P):
            fw = f * w[p:p + 1, :]
            n = jnp.sqrt(jnp.sum(fw * fw, axis=1, keepdims=True))
            cols.append(fw / jnp.maximum(n, 1e-12))
        y = jnp.concatenate(cols, axis=1)
        row = pid * _BLK
        y_s[pl.ds(row, _BLK), :] = y
        yt_s[:, pl.ds(row, _BLK)] = y.T

    @pl.when((pid >= _GRID) & (pid < 2 * _GRID))
    def _sim():
        row = (pid - _GRID) * _BLK
        y = y_s[pl.ds(row, _BLK), :]
        s = jax.lax.dot_general(
            y, yt_s[...], (((1,), (0,)), ((), ())),
            preferred_element_type=jnp.float32) * (1.0 / _P)
        s_s[pl.ds(row, _BLK), :] = s

        def body(_, carry):
            lo, hi = carry
            mid = (lo + hi) * 0.5
            cnt = jnp.count_nonzero(s >= mid, axis=1, keepdims=True)
            pred = cnt >= _TOPK
            return jnp.where(pred, mid, lo), jnp.where(pred, hi, mid)

        lo, hi = jax.lax.fori_loop(
            0, _BISECT_ITERS, body,
            (jnp.full((_BLK, 1), -1.001, jnp.float32),
             jnp.full((_BLK, 1), 1.001, jnp.float32)),
            unroll=_BISECT_ITERS)
        tc_s[pl.ds(row, _BLK), :] = lo
        tr_s[:, pl.ds(row, _BLK)] = lo.T

    @pl.when(pid >= 2 * _GRID)
    def _mask():
        row = (pid - 2 * _GRID) * _BLK
        s = s_s[pl.ds(row, _BLK), :]
        ti = tc_s[pl.ds(row, _BLK), :]
        tj = tr_s[...]
        keep = (s >= ti).astype(jnp.float32) + (s >= tj).astype(jnp.float32)
        o_ref[...] = s * keep * 0.5


@jax.jit
def kernel(features, weight_tensor):
    return pl.pallas_call(
        _fused_kernel,
        grid=(3 * _GRID,),
        in_specs=[
            pl.BlockSpec((_BLK, _D), lambda i: (jnp.minimum(i, _GRID - 1), 0)),
            pl.BlockSpec((_P, _D), lambda i: (0, 0)),
        ],
        out_specs=pl.BlockSpec(
            (_BLK, _N), lambda i: (jnp.maximum(i - 2 * _GRID, 0), 0)),
        out_shape=jax.ShapeDtypeStruct((_N, _N), jnp.float32),
        scratch_shapes=[
            pltpu.VMEM((_N, _PD), jnp.float32),
            pltpu.VMEM((_PD, _N), jnp.float32),
            pltpu.VMEM((_N, _N), jnp.float32),
            pltpu.VMEM((_N, 1), jnp.float32),
            pltpu.VMEM((1, _N), jnp.float32),
        ],
    )(features, weight_tensor)
